# Initial kernel scaffold; baseline (speedup 1.0000x reference)
import jax
import jax.numpy as jnp
from jax.experimental import pallas as pl


def kernel(image_embeddings, reference_feats, orig_h, orig_w):
    raise NotImplementedError("write your pallas kernel here")


# speedup vs baseline: 2.6665x; 2.6665x over previous
import functools

import numpy as np
import jax
import jax.numpy as jnp
from jax.experimental import pallas as pl
from jax.experimental.pallas import tpu as pltpu

_C = 256
_H = 64
_K = 16
_OH = 1024
_NPTS = 10
_HIGH = jax.lax.Precision.HIGHEST


def _resize_weights(in_size: int, out_size: int) -> np.ndarray:
    inv = in_size / out_size
    sample = (np.arange(out_size, dtype=np.float64) + 0.5) * inv - 0.5
    x = np.abs(sample[None, :] - np.arange(in_size, dtype=np.float64)[:, None])
    w = np.maximum(0.0, 1.0 - x)
    w = w / w.sum(axis=0, keepdims=True)
    return w.astype(np.float32)


_WX = _resize_weights(_H, _OH)
_WY = np.ascontiguousarray(_WX.T)


def _sim_body(t_ref, r_ref, sim_ref):
    rr = r_ref[...]
    rn = rr / (jnp.sqrt(jnp.sum(rr * rr, axis=1, keepdims=True)) + 1e-6)
    rnb = rn.astype(jnp.bfloat16)
    for j in range(16):
        ch = t_ref[:, pl.ds(j * 256, 256)]
        norm = jnp.sqrt(jnp.sum(ch * ch, axis=0, keepdims=True)) + 1e-6
        tnb = (ch / norm).astype(jnp.bfloat16)
        s = jax.lax.dot_general(rnb, tnb, (((1,), (0,)), ((), ())),
                                preferred_element_type=jnp.float32)
        sim_ref[:, pl.ds(j * 256, 256)] = s


def _sel_body(sim_ref, wx_ref, wy_ref, sc_ref, ix_ref,
              x_scr, map_scr, rm_scr, rmin_scr):
    t = pl.program_id(1)

    @pl.when(t == 0)
    def _():
        x_scr[...] = jax.lax.dot_general(
            sim_ref[...], wx_ref[...], (((1,), (0,)), ((), ())),
            preferred_element_type=jnp.float32, precision=_HIGH)

    mpt = jax.lax.dot_general(
        wy_ref[...], x_scr[...], (((1,), (0,)), ((), ())),
        preferred_element_type=jnp.float32, precision=_HIGH)
    map_scr[pl.ds(t * 128, 128), :] = mpt
    rm_scr[pl.ds(t, 1), :] = jnp.max(mpt, axis=1).reshape(1, 128)
    rmin_scr[pl.ds(t, 1), :] = jnp.min(mpt, axis=1).reshape(1, 128)

    @pl.when(t == 7)
    def _():
        ii = (jax.lax.broadcasted_iota(jnp.int32, (8, 128), 0) * 128 +
              jax.lax.broadcasted_iota(jnp.int32, (8, 128), 1))
        jc = jax.lax.broadcasted_iota(jnp.int32, (1, _OH), 1)
        lane = jax.lax.broadcasted_iota(jnp.int32, (1, 128), 1)
        big = jnp.int32(1 << 30)
        neg = jnp.float32(-jnp.inf)

        rm = rm_scr[...]
        rmin = rmin_scr[...]

        mn = jnp.min(rmin)
        rbg = jnp.min(jnp.where(rmin == mn, ii, big))
        rowbg = map_scr[pl.ds(rbg, 1), :]
        cbg = jnp.min(jnp.where(rowbg == mn, jc, big))
        idx_vec = jnp.where(lane == _NPTS, rbg * _OH + cbg,
                            jnp.zeros((1, 128), jnp.int32))
        sc_vec = jnp.zeros((1, 128), jnp.float32)

        for tt in range(_NPTS):
            m = jnp.max(rm)
            r = jnp.min(jnp.where(rm == m, ii, big))
            row = map_scr[pl.ds(r, 1), :]
            c = jnp.min(jnp.where(row == m, jc, big))
            sc_vec = jnp.where(lane == tt, m, sc_vec)
            idx_vec = jnp.where(lane == tt, r * _OH + c, idx_vec)
            nrow = jnp.where(jc == c, neg, row)
            map_scr[pl.ds(r, 1), :] = nrow
            rm = jnp.where(ii == r, jnp.max(nrow), rm)

        sc_ref[...] = sc_vec.reshape(1, 1, 128)
        ix_ref[...] = idx_vec.reshape(1, 1, 128)


@functools.partial(jax.jit, static_argnames=("interpret",))
def _run(target2, reference_feats, interpret=False):
    sim = pl.pallas_call(
        _sim_body,
        out_shape=jax.ShapeDtypeStruct((_K, _H * _H), jnp.float32),
        interpret=interpret,
    )(target2, reference_feats)

    sim2 = sim.reshape(_K * _H, _H)
    wx = jnp.asarray(_WX)
    wy = jnp.asarray(_WY)

    sc, ix = pl.pallas_call(
        _sel_body,
        grid=(_K, 8),
        in_specs=[
            pl.BlockSpec((_H, _H), lambda k, t: (k, 0)),
            pl.BlockSpec((_H, _OH), lambda k, t: (0, 0)),
            pl.BlockSpec((128, _H), lambda k, t: (t, 0)),
        ],
        out_specs=[
            pl.BlockSpec((1, 1, 128), lambda k, t: (k, 0, 0)),
            pl.BlockSpec((1, 1, 128), lambda k, t: (k, 0, 0)),
        ],
        out_shape=[
            jax.ShapeDtypeStruct((_K, 1, 128), jnp.float32),
            jax.ShapeDtypeStruct((_K, 1, 128), jnp.int32),
        ],
        scratch_shapes=[
            pltpu.VMEM((_H, _OH), jnp.float32),
            pltpu.VMEM((_OH, _OH), jnp.float32),
            pltpu.VMEM((8, 128), jnp.float32),
            pltpu.VMEM((8, 128), jnp.float32),
        ],
        interpret=interpret,
    )(sim2, wx, wy)
    return sc[:, 0, :], ix[:, 0, :]


def kernel(image_embeddings, reference_feats, orig_h, orig_w):
    target2 = image_embeddings.reshape(_C, _H * _H)
    sc, ix = _run(target2, reference_feats)
    scores = sc[:, :_NPTS]
    idx = ix[:, :_NPTS]
    xs = (idx % orig_w).astype(jnp.float32)
    ys = ((idx % (orig_h * orig_w)) // orig_w).astype(jnp.float32)
    points_scores = jnp.stack([xs, ys, scores], axis=-1)
    bgi = ix[:, _NPTS:_NPTS + 1]
    bg_x = (bgi % orig_w).astype(jnp.float32)
    bg_y = ((bgi % (orig_h * orig_w)) // orig_w).astype(jnp.float32)
    bg_coords = jnp.stack([bg_x, bg_y], axis=-1)
    return points_scores, bg_coords


# speedup vs baseline: 6.3944x; 2.3981x over previous
import functools

import numpy as np
import jax
import jax.numpy as jnp
from jax.experimental import pallas as pl
from jax.experimental.pallas import tpu as pltpu

_C = 256
_H = 64
_K = 16
_OH = 1024
_NPTS = 10
_NROWS = 12
_HIGH = jax.lax.Precision.HIGHEST


def _resize_weights(in_size: int, out_size: int) -> np.ndarray:
    inv = in_size / out_size
    sample = (np.arange(out_size, dtype=np.float64) + 0.5) * inv - 0.5
    x = np.abs(sample[None, :] - np.arange(in_size, dtype=np.float64)[:, None])
    w = np.maximum(0.0, 1.0 - x)
    w = w / w.sum(axis=0, keepdims=True)
    return w.astype(np.float32)


_WX = _resize_weights(_H, _OH)
_WY = np.ascontiguousarray(_WX.T)

_ECOLS = ([0, 23]
          + sum([[16 * m + 8, 16 * m + 23] for m in range(1, 62)], [])
          + [1000, 1023])
_ECOLS = _ECOLS + [0, 0]
_WXE = np.ascontiguousarray(_WX[:, _ECOLS])


def _sim_body(t_ref, r_ref, sim_ref):
    rr = r_ref[...]
    rn = rr / (jnp.sqrt(jnp.sum(rr * rr, axis=1, keepdims=True)) + 1e-6)
    rnb = rn.astype(jnp.bfloat16)
    for j in range(16):
        ch = t_ref[:, pl.ds(j * 256, 256)]
        norm = jnp.sqrt(jnp.sum(ch * ch, axis=0, keepdims=True)) + 1e-6
        tnb = (ch / norm).astype(jnp.bfloat16)
        s = jax.lax.dot_general(rnb, tnb, (((1,), (0,)), ((), ())),
                                preferred_element_type=jnp.float32)
        sim_ref[:, pl.ds(j * 256, 256)] = s


def _dot(a, b):
    return jax.lax.dot_general(a, b, (((1,), (0,)), ((), ())),
                               preferred_element_type=jnp.float32,
                               precision=_HIGH)


def _sel_body(sim_ref, wx_ref, wxe_ref, wy_ref, sc_ref, ix_ref,
              x_scr, xe_scr, cand_scr, rm_scr, rmin_scr):
    k = pl.program_id(0)
    t = pl.program_id(1)

    @pl.when(t == 0)
    def _():
        x_scr[pl.ds(_H * k, _H), :] = _dot(sim_ref[...], wx_ref[...])
        xe_scr[pl.ds(_H * k, _H), :] = _dot(sim_ref[...], wxe_ref[...])

    mpt = _dot(wy_ref[pl.ds(128 * t, 128), :],
               xe_scr[pl.ds(_H * k, _H), :])
    rm_scr[pl.ds(8 * k + t, 1), :] = jnp.max(mpt, axis=1).reshape(1, 128)
    rmin_scr[pl.ds(8 * k + t, 1), :] = jnp.min(mpt, axis=1).reshape(1, 128)

    @pl.when(jnp.logical_and(k == _K - 1, t == 7))
    def _():
        big = jnp.int32(1 << 30)
        neg = jnp.float32(-jnp.inf)
        riota = (jax.lax.broadcasted_iota(jnp.int32, (_K, 8, 128), 1) * 128 +
                 jax.lax.broadcasted_iota(jnp.int32, (_K, 8, 128), 2))
        slot = jax.lax.broadcasted_iota(jnp.int32, (1, 16), 1)

        rm = rm_scr[...].reshape(_K, 8, 128)
        r_sel = jnp.zeros((_K, 16), jnp.int32)
        rv0 = None
        for i in range(_NROWS):
            mv = jnp.max(jnp.max(rm, axis=2), axis=1)[:, None, None]
            rv = jnp.min(jnp.min(jnp.where(rm == mv, riota, big), axis=2),
                         axis=1)[:, None]
            if i == 0:
                rv0 = rv
            r_sel = jnp.where(slot == i, rv, r_sel)
            rm = jnp.where(riota == rv[:, :, None], neg, rm)
        rmin = rmin_scr[...].reshape(_K, 8, 128)
        mnv = jnp.min(jnp.min(rmin, axis=2), axis=1)[:, None, None]
        rbg = jnp.min(jnp.min(jnp.where(rmin == mnv, riota, big), axis=2),
                      axis=1)[:, None]
        r_sel = jnp.where(slot == _NROWS, rbg, r_sel)
        r_sel = jnp.where(slot > _NROWS, rv0, r_sel)

        col3 = jax.lax.broadcasted_iota(jnp.int32, (_K, 16, _OH), 2)
        oh3 = jnp.where(col3 == r_sel[:, :, None], jnp.float32(1.0),
                        jnp.float32(0.0))
        for kk in range(_K):
            rows_w = _dot(oh3[kk], wy_ref[...])
            cand_scr[pl.ds(16 * kk, 16), :] = _dot(
                rows_w, x_scr[pl.ds(_H * kk, _H), :])

        cand = cand_scr[...].reshape(_K, 16, _OH)
        gidx = r_sel[:, :, None] * _OH + col3
        lane = jax.lax.broadcasted_iota(jnp.int32, (1, 128), 1)

        bgrow = cand[:, _NROWS, :]
        bgg = gidx[:, _NROWS, :]
        mnb = jnp.min(bgrow, axis=1, keepdims=True)
        gbg = jnp.min(jnp.where(bgrow == mnb, bgg, big), axis=1,
                      keepdims=True)
        ix_mat = jnp.where(lane == _NPTS, gbg,
                           jnp.zeros((_K, 128), jnp.int32))
        sc_mat = jnp.zeros((_K, 128), jnp.float32)

        for tt in range(_NPTS):
            m2 = jnp.max(cand, axis=2)
            m = jnp.max(m2, axis=1)[:, None, None]
            g3 = jnp.where(cand == m, gidx, big)
            g2 = jnp.min(g3, axis=2)
            g = jnp.min(g2, axis=1)[:, None]
            sc_mat = jnp.where(lane == tt, m[:, :, 0], sc_mat)
            ix_mat = jnp.where(lane == tt, g, ix_mat)
            cand = jnp.where(gidx == g[:, :, None], neg, cand)

        sc_ref[...] = sc_mat
        ix_ref[...] = ix_mat


@functools.partial(jax.jit, static_argnames=("interpret",))
def _run(target2, reference_feats, interpret=False):
    sim = pl.pallas_call(
        _sim_body,
        out_shape=jax.ShapeDtypeStruct((_K, _H * _H), jnp.float32),
        interpret=interpret,
    )(target2, reference_feats)

    sim2 = sim.reshape(_K * _H, _H)

    sc, ix = pl.pallas_call(
        _sel_body,
        grid=(_K, 8),
        in_specs=[
            pl.BlockSpec((_H, _H), lambda k, t: (k, 0)),
            pl.BlockSpec((_H, _OH), lambda k, t: (0, 0)),
            pl.BlockSpec((_H, 128), lambda k, t: (0, 0)),
            pl.BlockSpec((_OH, _H), lambda k, t: (0, 0)),
        ],
        out_specs=[
            pl.BlockSpec((_K, 128), lambda k, t: (0, 0)),
            pl.BlockSpec((_K, 128), lambda k, t: (0, 0)),
        ],
        out_shape=[
            jax.ShapeDtypeStruct((_K, 128), jnp.float32),
            jax.ShapeDtypeStruct((_K, 128), jnp.int32),
        ],
        scratch_shapes=[
            pltpu.VMEM((_K * _H, _OH), jnp.float32),
            pltpu.VMEM((_K * _H, 128), jnp.float32),
            pltpu.VMEM((_K * 16, _OH), jnp.float32),
            pltpu.VMEM((_K * 8, 128), jnp.float32),
            pltpu.VMEM((_K * 8, 128), jnp.float32),
        ],
        interpret=interpret,
    )(sim2, jnp.asarray(_WX), jnp.asarray(_WXE), jnp.asarray(_WY))
    return sc, ix


def kernel(image_embeddings, reference_feats, orig_h, orig_w):
    target2 = image_embeddings.reshape(_C, _H * _H)
    sc, ix = _run(target2, reference_feats)
    scores = sc[:, :_NPTS]
    idx = ix[:, :_NPTS]
    xs = (idx % orig_w).astype(jnp.float32)
    ys = ((idx % (orig_h * orig_w)) // orig_w).astype(jnp.float32)
    points_scores = jnp.stack([xs, ys, scores], axis=-1)
    bgi = ix[:, _NPTS:_NPTS + 1]
    bg_x = (bgi % orig_w).astype(jnp.float32)
    bg_y = ((bgi % (orig_h * orig_w)) // orig_w).astype(jnp.float32)
    bg_coords = jnp.stack([bg_x, bg_y], axis=-1)
    return points_scores, bg_coords


# speedup vs baseline: 11.4713x; 1.7940x over previous
import functools

import numpy as np
import jax
import jax.numpy as jnp
from jax.experimental import pallas as pl
from jax.experimental.pallas import tpu as pltpu

_C = 256
_H = 64
_K = 16
_OH = 1024
_NPTS = 10
_NROWS = 12
_HIGH = jax.lax.Precision.HIGHEST


def _resize_weights(in_size: int, out_size: int) -> np.ndarray:
    inv = in_size / out_size
    sample = (np.arange(out_size, dtype=np.float64) + 0.5) * inv - 0.5
    x = np.abs(sample[None, :] - np.arange(in_size, dtype=np.float64)[:, None])
    w = np.maximum(0.0, 1.0 - x)
    w = w / w.sum(axis=0, keepdims=True)
    return w.astype(np.float32)


_WX = _resize_weights(_H, _OH)
_WY = np.ascontiguousarray(_WX.T)

_ECOLS = ([0, 23]
          + sum([[16 * m + 8, 16 * m + 23] for m in range(1, 62)], [])
          + [1000, 1023])
_ECOLS = _ECOLS + [0, 0]
_WXE = np.ascontiguousarray(_WX[:, _ECOLS])


def _sim_body(t_ref, r_ref, sim_ref):
    rr = r_ref[...]
    rn = rr / (jnp.sqrt(jnp.sum(rr * rr, axis=1, keepdims=True)) + 1e-6)
    rnb = rn.astype(jnp.bfloat16)
    for j in range(16):
        ch = t_ref[:, pl.ds(j * 256, 256)]
        norm = jnp.sqrt(jnp.sum(ch * ch, axis=0, keepdims=True)) + 1e-6
        tnb = (ch / norm).astype(jnp.bfloat16)
        s = jax.lax.dot_general(rnb, tnb, (((1,), (0,)), ((), ())),
                                preferred_element_type=jnp.float32)
        sim_ref[:, pl.ds(j * 256, 256)] = s


def _dot(a, b):
    return jax.lax.dot_general(a, b, (((1,), (0,)), ((), ())),
                               preferred_element_type=jnp.float32,
                               precision=_HIGH)


def _sel_body(sim_ref, wx_ref, wxe_ref, wy_ref, sc_ref, ix_ref,
              x_scr, xe_scr, cand_scr, rm_scr):
    big = jnp.int32(1 << 30)
    neg = jnp.float32(-jnp.inf)

    x_scr[...] = _dot(sim_ref[...], wx_ref[...])
    xe_scr[...] = _dot(sim_ref[...], wxe_ref[...])

    rowio = jax.lax.broadcasted_iota(jnp.int32, (_OH, 128), 0)
    bg_rows = []
    for k in range(_K):
        tile = _dot(wy_ref[...], xe_scr[pl.ds(_H * k, _H), :])
        rm_scr[pl.ds(k, 1), :] = jnp.max(tile, axis=1).reshape(1, _OH)
        mn = jnp.min(tile)
        bg_rows.append(jnp.min(jnp.where(tile == mn, rowio, big)))

    rbg = jnp.stack(bg_rows)[:, None]
    riota = jax.lax.broadcasted_iota(jnp.int32, (_K, _OH), 1)
    slot = jax.lax.broadcasted_iota(jnp.int32, (1, 16), 1)

    rm = rm_scr[...]
    r_sel = jnp.zeros((_K, 16), jnp.int32)
    rv0 = None
    for i in range(_NROWS):
        mv = jnp.max(rm, axis=1, keepdims=True)
        rv = jnp.min(jnp.where(rm == mv, riota, big), axis=1, keepdims=True)
        if i == 0:
            rv0 = rv
        r_sel = jnp.where(slot == i, rv, r_sel)
        rm = jnp.where(riota == rv, neg, rm)
    r_sel = jnp.where(slot == _NROWS, rbg, r_sel)
    r_sel = jnp.where(slot > _NROWS, rv0, r_sel)

    col3 = jax.lax.broadcasted_iota(jnp.int32, (_K, 16, _OH), 2)
    oh3 = jnp.where(col3 == r_sel[:, :, None], jnp.float32(1.0),
                    jnp.float32(0.0))
    for kk in range(_K):
        rows_w = _dot(oh3[kk], wy_ref[...])
        cand_scr[pl.ds(16 * kk, 16), :] = _dot(
            rows_w, x_scr[pl.ds(_H * kk, _H), :])

    cand = cand_scr[...].reshape(_K, 16, _OH)
    gidx = r_sel[:, :, None] * _OH + col3
    lane = jax.lax.broadcasted_iota(jnp.int32, (1, 128), 1)

    bgrow = cand[:, _NROWS, :]
    bgg = gidx[:, _NROWS, :]
    mnb = jnp.min(bgrow, axis=1, keepdims=True)
    gbg = jnp.min(jnp.where(bgrow == mnb, bgg, big), axis=1,
                  keepdims=True)
    ix_mat = jnp.where(lane == _NPTS, gbg,
                       jnp.zeros((_K, 128), jnp.int32))
    sc_mat = jnp.zeros((_K, 128), jnp.float32)

    for tt in range(_NPTS):
        m2 = jnp.max(cand, axis=2)
        m = jnp.max(m2, axis=1)[:, None, None]
        g3 = jnp.where(cand == m, gidx, big)
        g2 = jnp.min(g3, axis=2)
        g = jnp.min(g2, axis=1)[:, None]
        sc_mat = jnp.where(lane == tt, m[:, :, 0], sc_mat)
        ix_mat = jnp.where(lane == tt, g, ix_mat)
        cand = jnp.where(gidx == g[:, :, None], neg, cand)

    sc_ref[...] = sc_mat
    ix_ref[...] = ix_mat


@functools.partial(jax.jit, static_argnames=("interpret",))
def _run(target2, reference_feats, interpret=False):
    sim = pl.pallas_call(
        _sim_body,
        out_shape=jax.ShapeDtypeStruct((_K, _H * _H), jnp.float32),
        interpret=interpret,
    )(target2, reference_feats)

    sim2 = sim.reshape(_K * _H, _H)

    sc, ix = pl.pallas_call(
        _sel_body,
        out_shape=[
            jax.ShapeDtypeStruct((_K, 128), jnp.float32),
            jax.ShapeDtypeStruct((_K, 128), jnp.int32),
        ],
        scratch_shapes=[
            pltpu.VMEM((_K * _H, _OH), jnp.float32),
            pltpu.VMEM((_K * _H, 128), jnp.float32),
            pltpu.VMEM((_K * 16, _OH), jnp.float32),
            pltpu.VMEM((_K, _OH), jnp.float32),
        ],
        interpret=interpret,
    )(sim2, jnp.asarray(_WX), jnp.asarray(_WXE), jnp.asarray(_WY))
    return sc, ix


def kernel(image_embeddings, reference_feats, orig_h, orig_w):
    target2 = image_embeddings.reshape(_C, _H * _H)
    sc, ix = _run(target2, reference_feats)
    scores = sc[:, :_NPTS]
    idx = ix[:, :_NPTS]
    xs = (idx % orig_w).astype(jnp.float32)
    ys = ((idx % (orig_h * orig_w)) // orig_w).astype(jnp.float32)
    points_scores = jnp.stack([xs, ys, scores], axis=-1)
    bgi = ix[:, _NPTS:_NPTS + 1]
    bg_x = (bgi % orig_w).astype(jnp.float32)
    bg_y = ((bgi % (orig_h * orig_w)) // orig_w).astype(jnp.float32)
    bg_coords = jnp.stack([bg_x, bg_y], axis=-1)
    return points_scores, bg_coords


# speedup vs baseline: 12.1635x; 1.0603x over previous
import functools

import numpy as np
import jax
import jax.numpy as jnp
from jax.experimental import pallas as pl
from jax.experimental.pallas import tpu as pltpu

_C = 256
_H = 64
_K = 16
_OH = 1024
_NPTS = 10
_NROWS = 12
_HIGH = jax.lax.Precision.HIGHEST


def _resize_weights(in_size: int, out_size: int) -> np.ndarray:
    inv = in_size / out_size
    sample = (np.arange(out_size, dtype=np.float64) + 0.5) * inv - 0.5
    x = np.abs(sample[None, :] - np.arange(in_size, dtype=np.float64)[:, None])
    w = np.maximum(0.0, 1.0 - x)
    w = w / w.sum(axis=0, keepdims=True)
    return w.astype(np.float32)


_WX = _resize_weights(_H, _OH)
_WY = np.ascontiguousarray(_WX.T)

_ECOLS = ([0, 23]
          + sum([[16 * m + 8, 16 * m + 23] for m in range(1, 62)], [])
          + [1000, 1023])
_ECOLS = _ECOLS + [0, 0]
_WXET = np.ascontiguousarray(_WX[:, _ECOLS].T)


def _sim_body(t_ref, r_ref, sim_ref):
    rr = r_ref[...]
    rn = rr / (jnp.sqrt(jnp.sum(rr * rr, axis=1, keepdims=True)) + 1e-6)
    rnb = rn.astype(jnp.bfloat16)
    for j in range(16):
        ch = t_ref[:, pl.ds(j * 256, 256)]
        norm = jnp.sqrt(jnp.sum(ch * ch, axis=0, keepdims=True)) + 1e-6
        tnb = (ch / norm).astype(jnp.bfloat16)
        s = jax.lax.dot_general(rnb, tnb, (((1,), (0,)), ((), ())),
                                preferred_element_type=jnp.float32)
        sim_ref[:, pl.ds(j * 256, 256)] = s


def _dot(a, b):
    return jax.lax.dot_general(a, b, (((1,), (0,)), ((), ())),
                               preferred_element_type=jnp.float32,
                               precision=_HIGH)


def _sel_body(sim_ref, simt_ref, wx_ref, wxet_ref, wy_ref, sc_ref, ix_ref,
              x_scr, cand_scr, rm_scr, rmin_scr):
    big = jnp.int32(1 << 30)
    neg = jnp.float32(-jnp.inf)

    x_scr[...] = _dot(sim_ref[...], wx_ref[...])

    for k in range(_K):
        xet = _dot(wxet_ref[...], simt_ref[pl.ds(_H * k, _H), :])
        tilet = _dot(xet, wx_ref[...])
        rm_scr[pl.ds(k, 1), :] = jnp.max(tilet, axis=0).reshape(1, _OH)
        rmin_scr[pl.ds(k, 1), :] = jnp.min(tilet, axis=0).reshape(1, _OH)

    riota = jax.lax.broadcasted_iota(jnp.int32, (_K, _OH), 1)
    slot = jax.lax.broadcasted_iota(jnp.int32, (1, 16), 1)

    rm = rm_scr[...]
    r_sel = jnp.zeros((_K, 16), jnp.int32)
    rv0 = None
    for i in range(_NROWS):
        mv = jnp.max(rm, axis=1, keepdims=True)
        rv = jnp.min(jnp.where(rm == mv, riota, big), axis=1, keepdims=True)
        if i == 0:
            rv0 = rv
        r_sel = jnp.where(slot == i, rv, r_sel)
        rm = jnp.where(riota == rv, neg, rm)
    rmin = rmin_scr[...]
    for i in range(2):
        mnv = jnp.min(rmin, axis=1, keepdims=True)
        rb = jnp.min(jnp.where(rmin == mnv, riota, big), axis=1,
                     keepdims=True)
        r_sel = jnp.where(slot == _NROWS + i, rb, r_sel)
        rmin = jnp.where(riota == rb, -neg, rmin)
    r_sel = jnp.where(slot > _NROWS + 1, rv0, r_sel)

    col3 = jax.lax.broadcasted_iota(jnp.int32, (_K, 16, _OH), 2)
    oh3 = jnp.where(col3 == r_sel[:, :, None], jnp.float32(1.0),
                    jnp.float32(0.0))
    for kk in range(_K):
        rows_w = _dot(oh3[kk], wy_ref[...])
        cand_scr[pl.ds(16 * kk, 16), :] = _dot(
            rows_w, x_scr[pl.ds(_H * kk, _H), :])

    cand = cand_scr[...].reshape(_K, 16, _OH)
    gidx = r_sel[:, :, None] * _OH + col3
    lane = jax.lax.broadcasted_iota(jnp.int32, (1, 128), 1)

    bgrow = cand[:, _NROWS:_NROWS + 2, :]
    bgg = gidx[:, _NROWS:_NROWS + 2, :]
    mnb = jnp.min(jnp.min(bgrow, axis=2), axis=1)[:, None, None]
    gbg = jnp.min(jnp.min(jnp.where(bgrow == mnb, bgg, big), axis=2),
                  axis=1)[:, None]
    ix_mat = jnp.where(lane == _NPTS, gbg,
                       jnp.zeros((_K, 128), jnp.int32))
    sc_mat = jnp.zeros((_K, 128), jnp.float32)

    for tt in range(_NPTS):
        m2 = jnp.max(cand, axis=2)
        m = jnp.max(m2, axis=1)[:, None, None]
        g3 = jnp.where(cand == m, gidx, big)
        g2 = jnp.min(g3, axis=2)
        g = jnp.min(g2, axis=1)[:, None]
        sc_mat = jnp.where(lane == tt, m[:, :, 0], sc_mat)
        ix_mat = jnp.where(lane == tt, g, ix_mat)
        cand = jnp.where(gidx == g[:, :, None], neg, cand)

    sc_ref[...] = sc_mat
    ix_ref[...] = ix_mat


@functools.partial(jax.jit, static_argnames=("interpret",))
def _run(target2, reference_feats, interpret=False):
    sim = pl.pallas_call(
        _sim_body,
        out_shape=jax.ShapeDtypeStruct((_K, _H * _H), jnp.float32),
        interpret=interpret,
    )(target2, reference_feats)

    sim2 = sim.reshape(_K * _H, _H)
    simt = sim.reshape(_K, _H, _H).transpose(0, 2, 1).reshape(_K * _H, _H)

    sc, ix = pl.pallas_call(
        _sel_body,
        out_shape=[
            jax.ShapeDtypeStruct((_K, 128), jnp.float32),
            jax.ShapeDtypeStruct((_K, 128), jnp.int32),
        ],
        scratch_shapes=[
            pltpu.VMEM((_K * _H, _OH), jnp.float32),
            pltpu.VMEM((_K * 16, _OH), jnp.float32),
            pltpu.VMEM((_K, _OH), jnp.float32),
            pltpu.VMEM((_K, _OH), jnp.float32),
        ],
        interpret=interpret,
    )(sim2, simt, jnp.asarray(_WX), jnp.asarray(_WXET), jnp.asarray(_WY))
    return sc, ix


def kernel(image_embeddings, reference_feats, orig_h, orig_w):
    target2 = image_embeddings.reshape(_C, _H * _H)
    sc, ix = _run(target2, reference_feats)
    scores = sc[:, :_NPTS]
    idx = ix[:, :_NPTS]
    xs = (idx % orig_w).astype(jnp.float32)
    ys = ((idx % (orig_h * orig_w)) // orig_w).astype(jnp.float32)
    points_scores = jnp.stack([xs, ys, scores], axis=-1)
    bgi = ix[:, _NPTS:_NPTS + 1]
    bg_x = (bgi % orig_w).astype(jnp.float32)
    bg_y = ((bgi % (orig_h * orig_w)) // orig_w).astype(jnp.float32)
    bg_coords = jnp.stack([bg_x, bg_y], axis=-1)
    return points_scores, bg_coords
